# Initial kernel scaffold; baseline (speedup 1.0000x reference)
import jax
import jax.numpy as jnp
from jax.experimental import pallas as pl


def kernel(x, pos, edge_index, batch, size, We1, be1, We2, be2, We3, be3, Wp1, bp1, Wp2, bp2, Wp3, bp3, Wl1, g1s, g1b, Wl2, g2s, g2b, Wl3, bl3):
    raise NotImplementedError("write your pallas kernel here")


# speedup vs baseline: 2.2762x; 2.2762x over previous
import functools

import jax
import jax.numpy as jnp
from jax import lax
from jax.experimental import pallas as pl
from jax.experimental.pallas import tpu as pltpu
from jax.experimental.pallas import tpu_sc as plsc

N = 10000
E = 320000
NB = 16
EPS = 1e-5
F = 64
H = 1024
NG = 16
GW = H // NG

NC, NS = 2, 16
NW = NC * NS
E_PER_W = E // NW
CH = 80
N_CH = E_PER_W // CH

BE = 1280
HW = 16


def _encoder_body(x_ref, pos_ref, we1, be1, we2, be2, we3, be3,
                  wp1, bp1, wp2, bp2, wp3, bp3, feat_ref):
    x = x_ref[...]
    h = jnp.maximum(x * we1[...] + be1[...], 0.0)
    h = jnp.maximum(jnp.dot(h, we2[...], preferred_element_type=jnp.float32)
                    + be2[...], 0.0)
    xe = jnp.dot(h, we3[...], preferred_element_type=jnp.float32) + be3[...]
    p = jnp.maximum(jnp.dot(pos_ref[...], wp1[...],
                            preferred_element_type=jnp.float32) + bp1[...], 0.0)
    p = jnp.maximum(jnp.dot(p, wp2[...], preferred_element_type=jnp.float32)
                    + bp2[...], 0.0)
    pe = jnp.dot(p, wp3[...], preferred_element_type=jnp.float32) + bp3[...]
    feat_ref[...] = jnp.concatenate([xe, pe], axis=1)


def _gather_body(feat_hbm, src_hbm, dst_hbm, xi_hbm, xj_hbm,
                 idx_v, rows_v, sem):
    wid = lax.axis_index("s") * NC + lax.axis_index("c")
    base = wid * E_PER_W

    def step(i, carry):
        off = base + i * CH
        pltpu.sync_copy(dst_hbm.at[pl.ds(off, CH)], idx_v)
        pltpu.async_copy(feat_hbm.at[idx_v], rows_v, sem).wait()
        pltpu.sync_copy(rows_v, xi_hbm.at[pl.ds(off, CH)])
        pltpu.sync_copy(src_hbm.at[pl.ds(off, CH)], idx_v)
        pltpu.async_copy(feat_hbm.at[idx_v], rows_v, sem).wait()
        pltpu.sync_copy(rows_v, xj_hbm.at[pl.ds(off, CH)])
        return carry

    lax.fori_loop(0, N_CH, step, 0)


def _edge_mlp_body(xi_ref, xj_ref, w1d, w1b, w2, w3, bl3,
                   g1s, g1b, g2s, g2b, gmat, emat, out_ref):
    f32 = jnp.float32
    h = (jnp.dot(xi_ref[...], w1d[...], preferred_element_type=f32)
         + jnp.dot(xj_ref[...], w1b[...], preferred_element_type=f32))
    mu = jnp.dot(h, gmat[...], preferred_element_type=f32)
    m2 = jnp.dot(h * h, gmat[...], preferred_element_type=f32)
    inv = lax.rsqrt(m2 - mu * mu + EPS)
    muf = jnp.dot(mu, emat[...], preferred_element_type=f32)
    invf = jnp.dot(inv, emat[...], preferred_element_type=f32)
    h = ((h - muf) * invf) * g1s[...] + g1b[...]
    h = jnp.maximum(h, 0.0)
    h = jnp.dot(h, w2[...], preferred_element_type=f32)
    mu = jnp.dot(h, gmat[...], preferred_element_type=f32)
    m2 = jnp.dot(h * h, gmat[...], preferred_element_type=f32)
    inv = lax.rsqrt(m2 - mu * mu + EPS)
    muf = jnp.dot(mu, emat[...], preferred_element_type=f32)
    invf = jnp.dot(inv, emat[...], preferred_element_type=f32)
    h = ((h - muf) * invf) * g2s[...] + g2b[...]
    h = jnp.maximum(h, 0.0)
    out_ref[...] = jnp.dot(h, w3[...], preferred_element_type=f32) + bl3[...]


def _scatter_body(h3_hbm, dst_hbm, zeros_hbm, out_hbm, idx_v, rows_v, shared):
    cid = lax.axis_index("c")
    sid = lax.axis_index("s")
    wid = sid * NC + cid
    base = wid * E_PER_W

    @pl.when(sid == 0)
    def _():
        pltpu.sync_copy(zeros_hbm, shared)

    plsc.subcore_barrier()

    def step(i, carry):
        off = base + i * CH
        pltpu.sync_copy(dst_hbm.at[pl.ds(off, CH)], idx_v)
        pltpu.sync_copy(h3_hbm.at[pl.ds(off, CH)], rows_v)
        pltpu.sync_copy(rows_v, shared.at[idx_v], add=True)
        return carry

    lax.fori_loop(0, N_CH, step, 0)
    plsc.subcore_barrier()

    @pl.when(sid == 0)
    def _():
        pltpu.sync_copy(shared, out_hbm.at[cid])


def _pool_body(parts_ref, batch_ref, q_ref):
    ns = (parts_ref[0] + parts_ref[1])[:, :4]
    b = batch_ref[...]
    neg = jnp.float32(-jnp.inf)
    cols = []
    for g in range(NB):
        vals = jnp.where(b == g, ns, neg)
        cols.append(jnp.max(vals, axis=0, keepdims=True))
    q = jnp.concatenate(cols, axis=0)
    q_ref[...] = jnp.where(jnp.isfinite(q), q, 0.0)


def kernel(x, pos, edge_index, batch, size,
           We1, be1, We2, be2, We3, be3,
           Wp1, bp1, Wp2, bp2, Wp3, bp3,
           Wl1, g1s, g1b, Wl2, g2s, g2b, Wl3, bl3):
    f32 = jnp.float32
    src = edge_index[0]
    dst = edge_index[1]

    r = lambda v: v.reshape(1, -1)
    feature = pl.pallas_call(
        _encoder_body,
        out_shape=jax.ShapeDtypeStruct((N, F), f32),
    )(x, pos, r(We1), r(be1), We2, r(be2), We3, r(be3),
      Wp1, r(bp1), Wp2, r(bp2), Wp3, r(bp3))

    mesh = plsc.VectorSubcoreMesh(core_axis_name="c", subcore_axis_name="s")
    xi, xj = pl.kernel(
        _gather_body,
        out_type=[jax.ShapeDtypeStruct((E, F), f32),
                  jax.ShapeDtypeStruct((E, F), f32)],
        mesh=mesh,
        scratch_types=[pltpu.VMEM((CH,), jnp.int32),
                       pltpu.VMEM((CH, F), f32),
                       pltpu.SemaphoreType.DMA],
        compiler_params=pltpu.CompilerParams(use_tc_tiling_on_sc=False),
    )(feature, src, dst)

    w1a, w1b = Wl1[:F], Wl1[F:]
    w1d = w1a - w1b
    w3p = jnp.pad(Wl3, ((0, 0), (0, HW - 4)))
    bl3p = jnp.pad(bl3, (0, HW - 4))
    gidx = jnp.arange(H, dtype=jnp.int32) // GW
    gmat = (jax.nn.one_hot(gidx, NG, dtype=f32) / GW)
    emat = jax.nn.one_hot(gidx, NG, dtype=f32).T

    h3 = pl.pallas_call(
        _edge_mlp_body,
        grid=(E // BE,),
        in_specs=[
            pl.BlockSpec((BE, F), lambda i: (i, 0)),
            pl.BlockSpec((BE, F), lambda i: (i, 0)),
            pl.BlockSpec((F, H), lambda i: (0, 0)),
            pl.BlockSpec((F, H), lambda i: (0, 0)),
            pl.BlockSpec((H, H), lambda i: (0, 0)),
            pl.BlockSpec((H, HW), lambda i: (0, 0)),
            pl.BlockSpec((1, HW), lambda i: (0, 0)),
            pl.BlockSpec((1, H), lambda i: (0, 0)),
            pl.BlockSpec((1, H), lambda i: (0, 0)),
            pl.BlockSpec((1, H), lambda i: (0, 0)),
            pl.BlockSpec((1, H), lambda i: (0, 0)),
            pl.BlockSpec((H, NG), lambda i: (0, 0)),
            pl.BlockSpec((NG, H), lambda i: (0, 0)),
        ],
        out_specs=pl.BlockSpec((BE, HW), lambda i: (i, 0)),
        out_shape=jax.ShapeDtypeStruct((E, HW), f32),
        compiler_params=pltpu.CompilerParams(
            dimension_semantics=("arbitrary",)),
    )(xi, xj, w1d, w1b, Wl2, w3p, r(bl3p),
      r(g1s), r(g1b), r(g2s), r(g2b), gmat, emat)

    parts = pl.kernel(
        _scatter_body,
        out_type=jax.ShapeDtypeStruct((NC, N, HW), f32),
        mesh=mesh,
        scratch_types=[pltpu.VMEM((CH,), jnp.int32),
                       pltpu.VMEM((CH, HW), f32),
                       pltpu.VMEM_SHARED((N, HW), f32)],
        compiler_params=pltpu.CompilerParams(use_tc_tiling_on_sc=False),
    )(h3, dst, jnp.zeros((N, HW), f32))

    q = pl.pallas_call(
        _pool_body,
        out_shape=jax.ShapeDtypeStruct((NB, 4), f32),
    )(parts, batch.reshape(N, 1))

    return q, feature
